# baseline (speedup 1.0000x reference)
import functools

import jax
import jax.numpy as jnp
from jax import lax
from jax.experimental import pallas as pl
from jax.experimental.pallas import tpu as pltpu
from jax.experimental.pallas import tpu_sc as plsc

H = 768
NW = 32
CG = 64
NR = 2
CHUNKS = (8, 24, 24, 8)
EPS = 1e-12


def _sc_gather(ids_flat, word_emb):
    TOK = ids_flat.shape[0]
    TPW = TOK // NW
    NIT = TPW // CG

    mesh = plsc.VectorSubcoreMesh(core_axis_name="c", subcore_axis_name="s")

    @functools.partial(
        pl.kernel,
        out_type=jax.ShapeDtypeStruct((TOK, H), jnp.float32),
        mesh=mesh,
        scratch_types=[
            pltpu.VMEM((TPW,), jnp.int32),
            pltpu.VMEM((NR * CG, H), jnp.float32),
            pltpu.SemaphoreType.DMA((NR,)),
            pltpu.SemaphoreType.DMA((NR,)),
        ],
    )
    def gather_k(ids_h, wemb_h, st_h, idx_v, rows, gsem, osem):
        w = lax.axis_index("s") * 2 + lax.axis_index("c")
        t0 = w * TPW
        pltpu.sync_copy(ids_h.at[pl.ds(t0, TPW)], idx_v)

        def gdesc(j):
            p = lax.rem(j, NR)
            return pltpu.make_async_copy(
                wemb_h.at[idx_v.at[pl.ds(j * CG, CG)]],
                rows.at[pl.ds(p * CG, CG)], gsem.at[p])

        def odesc(j):
            p = lax.rem(j, NR)
            return pltpu.make_async_copy(
                rows.at[pl.ds(p * CG, CG)],
                st_h.at[pl.ds(t0 + j * CG, CG)], osem.at[p])

        gdesc(jnp.int32(0)).start()

        def body(j, c):
            @pl.when(j + 1 < NIT)
            def _():
                @pl.when(j >= NR - 1)
                def _():
                    odesc(j + 1 - NR).wait()
                gdesc(j + 1).start()

            gdesc(j).wait()
            odesc(j).start()
            return c

        lax.fori_loop(0, NIT, body, 0)
        for jj in range(NIT - NR, NIT):
            odesc(jnp.int32(jj)).wait()

    return gather_k(ids_flat, word_emb)


def _tc_add_ln(prev, staged, ttf3, pos_emb, type_emb, gamma2, beta2,
               BC, S, b0, BTOT):
    def body(*refs):
        if b0 > 0:
            refs = refs[1:]
        st_ref, tt_ref, pos_ref, ty_ref, gam_ref, bet_ref, o_ref = refs
        x = st_ref[...] + pos_ref[...]
        t0 = ty_ref[0, :]
        d = ty_ref[1, :] - t0
        tt = tt_ref[0, 0, :]
        x = x + t0[None, :] + tt[:, None] * d[None, :]
        m = jnp.mean(x, axis=-1, keepdims=True)
        xc = x - m
        var = jnp.mean(xc * xc, axis=-1, keepdims=True)
        inv = lax.rsqrt(var + EPS)
        o_ref[...] = xc * inv * gam_ref[...] + bet_ref[...]

    in_specs = [
        pl.BlockSpec((S, H), lambda i: (i, 0)),
        pl.BlockSpec((1, 1, S), lambda i: (i, 0, 0)),
        pl.BlockSpec((S, H), lambda i: (0, 0)),
        pl.BlockSpec((2, H), lambda i: (0, 0)),
        pl.BlockSpec((1, H), lambda i: (0, 0)),
        pl.BlockSpec((1, H), lambda i: (0, 0)),
    ]
    args = [staged, ttf3, pos_emb, type_emb, gamma2, beta2]
    aliases = {}
    if b0 > 0:
        in_specs = [pl.BlockSpec(memory_space=pl.ANY)] + in_specs
        args = [prev] + args
        aliases = {0: 0}

    return pl.pallas_call(
        body,
        grid=(BC,),
        in_specs=in_specs,
        out_specs=pl.BlockSpec((S, H), lambda i: (i + b0, 0)),
        out_shape=jax.ShapeDtypeStruct((BTOT * S, H), jnp.float32),
        input_output_aliases=aliases,
        compiler_params=pltpu.CompilerParams(
            dimension_semantics=("arbitrary",)),
    )(*args)


def kernel(input_ids, token_type_ids, word_emb, pos_emb, type_emb, gamma, beta):
    B, S = input_ids.shape
    ids_flat = input_ids.reshape(-1)
    ttf3 = token_type_ids.astype(jnp.float32).reshape(B, 1, S)
    gamma2 = gamma.reshape(1, H)
    beta2 = beta.reshape(1, H)
    out = None
    b0 = 0
    for BC in CHUNKS:
        staged = _sc_gather(
            lax.slice_in_dim(ids_flat, b0 * S, (b0 + BC) * S), word_emb)
        out = _tc_add_ln(out, staged,
                         lax.slice_in_dim(ttf3, b0, b0 + BC),
                         pos_emb, type_emb, gamma2, beta2, BC, S, b0, B)
        b0 += BC
    return out.reshape(B, S, H)


# speedup vs baseline: 1.0159x; 1.0159x over previous
import functools

import jax
import jax.numpy as jnp
from jax import lax
from jax.experimental import pallas as pl
from jax.experimental.pallas import tpu as pltpu
from jax.experimental.pallas import tpu_sc as plsc

H = 768
NW = 32
CG = 64
NR = 2
CHUNKS = (16, 16, 16, 16)
EPS = 1e-12


def _sc_gather(ids_flat, word_emb):
    TOK = ids_flat.shape[0]
    TPW = TOK // NW
    NIT = TPW // CG

    mesh = plsc.VectorSubcoreMesh(core_axis_name="c", subcore_axis_name="s")

    @functools.partial(
        pl.kernel,
        out_type=jax.ShapeDtypeStruct((TOK, H), jnp.float32),
        mesh=mesh,
        scratch_types=[
            pltpu.VMEM((TPW,), jnp.int32),
            pltpu.VMEM((NR * CG, H), jnp.float32),
            pltpu.SemaphoreType.DMA((NR,)),
            pltpu.SemaphoreType.DMA((NR,)),
        ],
    )
    def gather_k(ids_h, wemb_h, st_h, idx_v, rows, gsem, osem):
        w = lax.axis_index("s") * 2 + lax.axis_index("c")
        t0 = w * TPW
        pltpu.sync_copy(ids_h.at[pl.ds(t0, TPW)], idx_v)

        def gdesc(j):
            p = lax.rem(j, NR)
            return pltpu.make_async_copy(
                wemb_h.at[idx_v.at[pl.ds(j * CG, CG)]],
                rows.at[pl.ds(p * CG, CG)], gsem.at[p])

        def odesc(j):
            p = lax.rem(j, NR)
            return pltpu.make_async_copy(
                rows.at[pl.ds(p * CG, CG)],
                st_h.at[pl.ds(t0 + j * CG, CG)], osem.at[p])

        gdesc(jnp.int32(0)).start()

        def body(j, c):
            @pl.when(j + 1 < NIT)
            def _():
                @pl.when(j >= NR - 1)
                def _():
                    odesc(j + 1 - NR).wait()
                gdesc(j + 1).start()

            gdesc(j).wait()
            odesc(j).start()
            return c

        lax.fori_loop(0, NIT, body, 0)
        for jj in range(NIT - NR, NIT):
            odesc(jnp.int32(jj)).wait()

    return gather_k(ids_flat, word_emb)


def _tc_add_ln(prev, staged, ttf3, pos_emb, type_emb, gamma2, beta2,
               BC, S, b0, BTOT):
    def body(*refs):
        if b0 > 0:
            refs = refs[1:]
        st_ref, tt_ref, pos_ref, ty_ref, gam_ref, bet_ref, o_ref = refs
        x = st_ref[...] + pos_ref[...]
        t0 = ty_ref[0, :]
        d = ty_ref[1, :] - t0
        tt = tt_ref[0, 0, :]
        x = x + t0[None, :] + tt[:, None] * d[None, :]
        m = jnp.mean(x, axis=-1, keepdims=True)
        xc = x - m
        var = jnp.mean(xc * xc, axis=-1, keepdims=True)
        inv = lax.rsqrt(var + EPS)
        o_ref[...] = xc * inv * gam_ref[...] + bet_ref[...]

    in_specs = [
        pl.BlockSpec((S, H), lambda i: (i, 0)),
        pl.BlockSpec((1, 1, S), lambda i: (i, 0, 0)),
        pl.BlockSpec((S, H), lambda i: (0, 0)),
        pl.BlockSpec((2, H), lambda i: (0, 0)),
        pl.BlockSpec((1, H), lambda i: (0, 0)),
        pl.BlockSpec((1, H), lambda i: (0, 0)),
    ]
    args = [staged, ttf3, pos_emb, type_emb, gamma2, beta2]
    aliases = {}
    if b0 > 0:
        in_specs = [pl.BlockSpec(memory_space=pl.ANY)] + in_specs
        args = [prev] + args
        aliases = {0: 0}

    return pl.pallas_call(
        body,
        grid=(BC,),
        in_specs=in_specs,
        out_specs=pl.BlockSpec((S, H), lambda i: (i + b0, 0)),
        out_shape=jax.ShapeDtypeStruct((BTOT * S, H), jnp.float32),
        input_output_aliases=aliases,
        compiler_params=pltpu.CompilerParams(
            dimension_semantics=("arbitrary",)),
    )(*args)


def kernel(input_ids, token_type_ids, word_emb, pos_emb, type_emb, gamma, beta):
    B, S = input_ids.shape
    ids_flat = input_ids.reshape(-1)
    ttf3 = token_type_ids.astype(jnp.float32).reshape(B, 1, S)
    gamma2 = gamma.reshape(1, H)
    beta2 = beta.reshape(1, H)
    out = None
    b0 = 0
    for BC in CHUNKS:
        staged = _sc_gather(
            lax.slice_in_dim(ids_flat, b0 * S, (b0 + BC) * S), word_emb)
        out = _tc_add_ln(out, staged,
                         lax.slice_in_dim(ttf3, b0, b0 + BC),
                         pos_emb, type_emb, gamma2, beta2, BC, S, b0, B)
        b0 += BC
    return out.reshape(B, S, H)
